# baseline (speedup 1.0000x reference)
import functools

import jax
import jax.numpy as jnp
from jax import lax
from jax.experimental import pallas as pl
from jax.experimental.pallas import tpu as pltpu
from jax.experimental.pallas import tpu_sc as plsc

F32 = jnp.float32
N_PTS = 16384
NBATCH = 16


MINV = 1.0 / (16 * 16384)


def _mega_body(x_ref, xt_ref, a1_ref, c1_ref, w2_ref, b2_ref, g2_ref, be2_ref,
               w3_ref, b3_ref, g3_ref, be3_ref, ctr_ref, prm_ref,
               mxh_ref, smh_ref, mxs_ref, sms_ref,
               su2_s, sq2_s, a2_s, c2_s, su3_s, sq3_s, a3_s, c3_s):
    p = pl.program_id(0)
    bi = pl.program_id(1)
    nb = pl.num_programs(1)
    xb = x_ref[0]
    h1 = jnp.maximum(
        jnp.dot(xb, a1_ref[...], preferred_element_type=F32) + c1_ref[...], 0.0)

    @pl.when(p == 0)
    def _():
        u = jnp.dot(h1, w2_ref[...], preferred_element_type=F32) + b2_ref[...]
        su = jnp.sum(u, axis=0, keepdims=True)
        sq = jnp.sum(u * u, axis=0, keepdims=True)

        @pl.when(bi == 0)
        def _():
            su2_s[...] = su
            sq2_s[...] = sq

        @pl.when(bi != 0)
        def _():
            su2_s[...] += su
            sq2_s[...] += sq

        @pl.when(bi == nb - 1)
        def _():
            m2 = su2_s[...] * MINV
            v2 = sq2_s[...] * MINV - m2 * m2
            s2 = g2_ref[...] / jnp.sqrt(v2 + 1e-5)
            a2_s[...] = w2_ref[...] * s2
            c2_s[...] = (b2_ref[...] - m2) * s2 + be2_ref[...]

    @pl.when(p == 1)
    def _():
        h2 = jnp.maximum(
            jnp.dot(h1, a2_s[...], preferred_element_type=F32) + c2_s[...], 0.0)
        u = jnp.dot(h2, w3_ref[...], preferred_element_type=F32) + b3_ref[...]
        su = jnp.sum(u, axis=0, keepdims=True)
        sq = jnp.sum(u * u, axis=0, keepdims=True)

        @pl.when(bi == 0)
        def _():
            su3_s[...] = su
            sq3_s[...] = sq

        @pl.when(bi != 0)
        def _():
            su3_s[...] += su
            sq3_s[...] += sq

        @pl.when(bi == nb - 1)
        def _():
            m3 = su3_s[...] * MINV
            v3 = sq3_s[...] * MINV - m3 * m3
            s3 = g3_ref[...] / jnp.sqrt(v3 + 1e-5)
            a3_s[...] = w3_ref[...] * s3
            c3_s[...] = (b3_ref[...] - m3) * s3 + be3_ref[...]

    @pl.when(p == 2)
    def _():
        h2 = jnp.maximum(
            jnp.dot(h1, a2_s[...], preferred_element_type=F32) + c2_s[...], 0.0)
        h3 = jnp.maximum(
            jnp.dot(h2, a3_s[...], preferred_element_type=F32) + c3_s[...], 0.0)
        mxh = jnp.max(h3, axis=0, keepdims=True)
        smh = jnp.sum(h3, axis=0, keepdims=True)

        xtb = xt_ref[0]
        xs = xtb[0:1]
        yv = xtb[1:2]
        zs = xtb[2:3]
        t = (yv - prm_ref[0, 0, 0]) / prm_ref[0, 0, 1] * 10.0
        bk = t.astype(jnp.int32)
        cx = jnp.zeros((1, N_PTS), F32)
        cz = jnp.zeros((1, N_PTS), F32)
        for k in range(11):
            mk = bk == k
            cx = jnp.where(mk, ctr_ref[0, 0, k], cx)
            cz = jnp.where(mk, ctr_ref[0, 0, 16 + k], cz)
        dx = xs - cx
        dz = zs - cz
        r = jnp.sqrt(dx * dx + dz * dz)
        inv = jnp.where(r > 0.0, 1.0 / r, 0.0)
        sn = dz * inv
        cs = jnp.where(r > 0.0, dx * inv, 1.0)

        def c3_(a, b, c):
            return jnp.concatenate(
                [jnp.reshape(a, (1, 1)), jnp.reshape(b, (1, 1)),
                 jnp.reshape(c, (1, 1))], axis=1)

        mxh_ref[...] = mxh[None]
        smh_ref[...] = smh[None]
        mxs_ref[...] = c3_(jnp.max(sn), jnp.max(cs), jnp.max(r))[None]
        sms_ref[...] = c3_(jnp.sum(sn), jnp.sum(cs), jnp.sum(r))[None]


def _p5_body(mxh_ref, mxs_ref, smh_ref, sms_ref,
             w1_ref, w2_ref, w3_ref, w4_ref,
             bp_ref, g_ref, b_ref, out_ref):
    invn = 1.0 / N_PTS
    o = (jnp.dot(mxh_ref[...], w1_ref[...], preferred_element_type=F32)
         + jnp.dot(mxs_ref[...], w2_ref[...], preferred_element_type=F32)
         + jnp.dot(smh_ref[...] * invn, w3_ref[...], preferred_element_type=F32)
         + jnp.dot(sms_ref[...] * invn, w4_ref[...], preferred_element_type=F32)
         + bp_ref[...])
    mu = jnp.mean(o, axis=1, keepdims=True)
    var = jnp.mean((o - mu) ** 2, axis=1, keepdims=True)
    out_ref[...] = (o - mu) / jnp.sqrt(var + 1e-5) * g_ref[...] + b_ref[...]


def _sc_body(xs_hbm, ys_hbm, zs_hbm, part_hbm, st_hbm, xv, yv, zv, acc, stv):
    c = lax.axis_index("c")
    s = lax.axis_index("s")
    w = s * 2 + c

    @pl.when(w < NBATCH)
    def _():
        base = w * N_PTS
        pltpu.sync_copy(xs_hbm.at[pl.ds(base, N_PTS)], xv)
        pltpu.sync_copy(ys_hbm.at[pl.ds(base, N_PTS)], yv)
        pltpu.sync_copy(zs_hbm.at[pl.ds(base, N_PTS)], zv)
        nit = N_PTS // 16
        big = jnp.float32(3.4e38)
        zero = jnp.zeros((16,), F32)
        init = (jnp.full((16,), big, F32), jnp.full((16,), -big, F32),
                zero, zero, zero, zero, zero, zero, zero, zero, zero)

        def body1(i, cr):
            mn, mx, sx, sy, sz, sxx, sxy, sxz, syy, syz, szz = cr
            off = i * 16
            xw = xv[pl.ds(off, 16)]
            yw = yv[pl.ds(off, 16)]
            zw = zv[pl.ds(off, 16)]
            return (jnp.minimum(mn, yw), jnp.maximum(mx, yw),
                    sx + xw, sy + yw, sz + zw,
                    sxx + xw * xw, sxy + xw * yw, sxz + xw * zw,
                    syy + yw * yw, syz + yw * zw, szz + zw * zw)

        st = lax.fori_loop(0, nit, body1, init)
        for i in range(11):
            stv[i] = st[i]
        stv[11] = zero
        pltpu.sync_copy(stv, st_hbm.at[w])

        mn_s = jnp.min(st[0])
        mx_s = jnp.max(st[1])
        den_s = mx_s - mn_s + 1e-6
        mn_v = jnp.full((16,), mn_s, F32)
        den_v = jnp.full((16,), den_s, F32)

        for i in range(3):
            for rr in range(16):
                acc[i, rr] = zero
        lanes = lax.iota(jnp.int32, 16)
        idx0 = jnp.zeros((16,), jnp.int32)
        idx1 = idx0 + 1
        idx2 = idx0 + 2
        ones_f = jnp.ones((16,), F32)

        def body2(i, carry):
            off = i * 16
            y16 = yv[pl.ds(off, 16)]
            x16 = xv[pl.ds(off, 16)]
            z16 = zv[pl.ds(off, 16)]
            t = (y16 - mn_v) / den_v * 10.0
            bk = t.astype(jnp.int32)
            plsc.addupdate_scatter(acc, [idx0, bk, lanes], ones_f)
            plsc.addupdate_scatter(acc, [idx1, bk, lanes], x16)
            plsc.addupdate_scatter(acc, [idx2, bk, lanes], z16)
            return carry

        lax.fori_loop(0, nit, body2, 0)
        pltpu.sync_copy(acc, part_hbm.at[w])


_SC_MESH = dict(
    mesh=plsc.VectorSubcoreMesh(core_axis_name="c", subcore_axis_name="s"),
    compiler_params=pltpu.CompilerParams(needs_layout_passes=False),
)


def _sc_hist(xs_f, ys_f, zs_f):
    fn = functools.partial(
        pl.kernel,
        out_type=[jax.ShapeDtypeStruct((NBATCH, 3, 16, 16), F32),
                  jax.ShapeDtypeStruct((NBATCH, 12, 16), F32)],
        scratch_types=[
            pltpu.VMEM((N_PTS,), F32),
            pltpu.VMEM((N_PTS,), F32),
            pltpu.VMEM((N_PTS,), F32),
            pltpu.VMEM((3, 16, 16), F32),
            pltpu.VMEM((12, 16), F32),
        ],
        **_SC_MESH,
    )(_sc_body)
    return fn(xs_f, ys_f, zs_f)


def kernel(x, W1, b1, g1, be1, W2, b2, g2, be2, W3, b3, g3, be3,
           Wp, bp, ln_g, ln_b):
    B, N, _ = x.shape
    M = B * N
    minv = 1.0 / M

    def full(shp):
        return pl.BlockSpec(shp, lambda p, b_: (0,) * len(shp))

    def sds(shp):
        return jax.ShapeDtypeStruct(shp, F32)

    xs_f = x[:, :, 0].reshape(M)
    ys_f = x[:, :, 1].reshape(M)
    zs_f = x[:, :, 2].reshape(M)
    part, st = _sc_hist(xs_f, ys_f, zs_f)

    ymn_b = jnp.min(st[:, 0, :], axis=1)
    ymx_b = jnp.max(st[:, 1, :], axis=1)
    den_b = ymx_b - ymn_b + 1e-6
    mom = jnp.sum(st[:, 2:11, :], axis=(0, 2))
    mu3 = mom[0:3] * minv
    cov = (jnp.stack([
        jnp.stack([mom[3], mom[4], mom[5]]),
        jnp.stack([mom[4], mom[6], mom[7]]),
        jnp.stack([mom[5], mom[7], mom[8]]),
    ]) * minv - mu3[:, None] * mu3[None, :])
    m1 = mu3 @ W1.T + b1
    v1 = jnp.einsum("jc,cd,jd->j", W1, cov, W1)
    s1 = g1 / jnp.sqrt(v1 + 1e-5)
    a1 = W1.T * s1[None]
    c1 = ((b1 - m1) * s1 + be1)[None]

    agg = jnp.sum(part, axis=3)
    cnt = agg[:, 0]
    safe = jnp.maximum(cnt, 1.0)
    pos = cnt > 0.0
    cx = jnp.where(pos, agg[:, 1] / safe, 0.0)
    cz = jnp.where(pos, agg[:, 2] / safe, 0.0)
    ctr = jnp.concatenate([cx, cz], axis=1).reshape(B, 1, 32)
    prm = jnp.stack([ymn_b, den_b], axis=1).reshape(B, 1, 2)

    xt = x.transpose(0, 2, 1)
    mxh, smh, mxs, sms = pl.pallas_call(
        _mega_body,
        grid=(3, B),
        in_specs=[pl.BlockSpec((1, N, 3), lambda p, b_: (b_, 0, 0)),
                  pl.BlockSpec((1, 3, N), lambda p, b_: (b_, 0, 0)),
                  full((3, 64)), full((1, 64)),
                  full((64, 128)), full((1, 128)),
                  full((1, 128)), full((1, 128)),
                  full((128, 256)), full((1, 256)),
                  full((1, 256)), full((1, 256)),
                  pl.BlockSpec((1, 1, 32), lambda p, b_: (b_, 0, 0),
                               memory_space=pltpu.SMEM),
                  pl.BlockSpec((1, 1, 2), lambda p, b_: (b_, 0, 0),
                               memory_space=pltpu.SMEM)],
        out_specs=[pl.BlockSpec((1, 1, 256), lambda p, b_: (b_, 0, 0)),
                   pl.BlockSpec((1, 1, 256), lambda p, b_: (b_, 0, 0)),
                   pl.BlockSpec((1, 1, 3), lambda p, b_: (b_, 0, 0)),
                   pl.BlockSpec((1, 1, 3), lambda p, b_: (b_, 0, 0))],
        out_shape=[sds((B, 1, 256)), sds((B, 1, 256)),
                   sds((B, 1, 3)), sds((B, 1, 3))],
        scratch_shapes=[pltpu.VMEM((1, 128), F32), pltpu.VMEM((1, 128), F32),
                        pltpu.VMEM((64, 128), F32), pltpu.VMEM((1, 128), F32),
                        pltpu.VMEM((1, 256), F32), pltpu.VMEM((1, 256), F32),
                        pltpu.VMEM((128, 256), F32), pltpu.VMEM((1, 256), F32)],
    )(x, xt, a1, c1, W2.T, b2[None], g2[None], be2[None],
      W3.T, b3[None], g3[None], be3[None], ctr, prm)

    wpt = Wp.T
    out = pl.pallas_call(
        _p5_body,
        out_shape=sds((B, 512)),
    )(mxh.reshape(B, 256), mxs.reshape(B, 3),
      smh.reshape(B, 256), sms.reshape(B, 3),
      wpt[0:256], wpt[256:259], wpt[259:515], wpt[515:518],
      bp[None], ln_g[None], ln_b[None])
    return out


# speedup vs baseline: 1.0088x; 1.0088x over previous
import functools

import jax
import jax.numpy as jnp
from jax import lax
from jax.experimental import pallas as pl
from jax.experimental.pallas import tpu as pltpu
from jax.experimental.pallas import tpu_sc as plsc

F32 = jnp.float32
N_PTS = 16384
NBATCH = 16


def _p2_body(x_ref, a1_ref, c1_ref, w_ref, b_ref, su_ref, sq_ref):
    bi = pl.program_id(0)
    xb = x_ref[0]
    h1 = jnp.maximum(
        jnp.dot(xb, a1_ref[...], preferred_element_type=F32) + c1_ref[...], 0.0)
    u = jnp.dot(h1, w_ref[...], preferred_element_type=F32) + b_ref[...]
    su = jnp.sum(u, axis=0, keepdims=True)
    sq = jnp.sum(u * u, axis=0, keepdims=True)

    @pl.when(bi == 0)
    def _():
        su_ref[...] = su
        sq_ref[...] = sq

    @pl.when(bi != 0)
    def _():
        su_ref[...] += su
        sq_ref[...] += sq


def _p3_body(x_ref, a1_ref, c1_ref, a2_ref, c2_ref, w_ref, b_ref,
             su_ref, sq_ref):
    bi = pl.program_id(0)
    xb = x_ref[0]
    h1 = jnp.maximum(
        jnp.dot(xb, a1_ref[...], preferred_element_type=F32) + c1_ref[...], 0.0)
    h2 = jnp.maximum(
        jnp.dot(h1, a2_ref[...], preferred_element_type=F32) + c2_ref[...], 0.0)
    u = jnp.dot(h2, w_ref[...], preferred_element_type=F32) + b_ref[...]
    su = jnp.sum(u, axis=0, keepdims=True)
    sq = jnp.sum(u * u, axis=0, keepdims=True)

    @pl.when(bi == 0)
    def _():
        su_ref[...] = su
        sq_ref[...] = sq

    @pl.when(bi != 0)
    def _():
        su_ref[...] += su
        sq_ref[...] += sq


def _p4_body(x_ref, xt_ref, a1_ref, c1_ref, a2_ref, c2_ref, a3_ref, c3_ref,
             ctr_ref, prm_ref,
             mxh_ref, smh_ref, mxs_ref, sms_ref):
    xb = x_ref[0]
    h1 = jnp.maximum(
        jnp.dot(xb, a1_ref[...], preferred_element_type=F32) + c1_ref[...], 0.0)
    h2 = jnp.maximum(
        jnp.dot(h1, a2_ref[...], preferred_element_type=F32) + c2_ref[...], 0.0)
    h3 = jnp.maximum(
        jnp.dot(h2, a3_ref[...], preferred_element_type=F32) + c3_ref[...], 0.0)
    mxh = jnp.max(h3, axis=0, keepdims=True)
    smh = jnp.sum(h3, axis=0, keepdims=True)

    xtb = xt_ref[0]
    xs = xtb[0:1]
    yv = xtb[1:2]
    zs = xtb[2:3]
    t = (yv - prm_ref[0, 0, 0]) / prm_ref[0, 0, 1] * 10.0
    bk = t.astype(jnp.int32)
    cx = jnp.zeros((1, N_PTS), F32)
    cz = jnp.zeros((1, N_PTS), F32)
    for k in range(11):
        mk = bk == k
        cx = jnp.where(mk, ctr_ref[0, 0, k], cx)
        cz = jnp.where(mk, ctr_ref[0, 0, 16 + k], cz)
    dx = xs - cx
    dz = zs - cz
    r = jnp.sqrt(dx * dx + dz * dz)
    inv = jnp.where(r > 0.0, 1.0 / r, 0.0)
    sn = dz * inv
    cs = jnp.where(r > 0.0, dx * inv, 1.0)

    def c3_(a, b, c):
        return jnp.concatenate(
            [jnp.reshape(a, (1, 1)), jnp.reshape(b, (1, 1)),
             jnp.reshape(c, (1, 1))], axis=1)

    mxs = c3_(jnp.max(sn), jnp.max(cs), jnp.max(r))
    sms = c3_(jnp.sum(sn), jnp.sum(cs), jnp.sum(r))

    mxh_ref[...] = mxh[None]
    smh_ref[...] = smh[None]
    mxs_ref[...] = mxs[None]
    sms_ref[...] = sms[None]


def _p5_body(mxh_ref, mxs_ref, smh_ref, sms_ref,
             w1_ref, w2_ref, w3_ref, w4_ref,
             bp_ref, g_ref, b_ref, out_ref):
    invn = 1.0 / N_PTS
    o = (jnp.dot(mxh_ref[...], w1_ref[...], preferred_element_type=F32)
         + jnp.dot(mxs_ref[...], w2_ref[...], preferred_element_type=F32)
         + jnp.dot(smh_ref[...] * invn, w3_ref[...], preferred_element_type=F32)
         + jnp.dot(sms_ref[...] * invn, w4_ref[...], preferred_element_type=F32)
         + bp_ref[...])
    mu = jnp.mean(o, axis=1, keepdims=True)
    var = jnp.mean((o - mu) ** 2, axis=1, keepdims=True)
    out_ref[...] = (o - mu) / jnp.sqrt(var + 1e-5) * g_ref[...] + b_ref[...]


def _sc_body(xs_hbm, ys_hbm, zs_hbm, part_hbm, st_hbm, xv, yv, zv, acc, stv):
    c = lax.axis_index("c")
    s = lax.axis_index("s")
    w = s * 2 + c

    @pl.when(w < NBATCH)
    def _():
        base = w * N_PTS
        pltpu.sync_copy(xs_hbm.at[pl.ds(base, N_PTS)], xv)
        pltpu.sync_copy(ys_hbm.at[pl.ds(base, N_PTS)], yv)
        pltpu.sync_copy(zs_hbm.at[pl.ds(base, N_PTS)], zv)
        nit = N_PTS // 16
        big = jnp.float32(3.4e38)
        zero = jnp.zeros((16,), F32)
        init = (jnp.full((16,), big, F32), jnp.full((16,), -big, F32),
                zero, zero, zero, zero, zero, zero, zero, zero, zero)

        def body1(i, cr):
            mn, mx, sx, sy, sz, sxx, sxy, sxz, syy, syz, szz = cr
            off = i * 16
            xw = xv[pl.ds(off, 16)]
            yw = yv[pl.ds(off, 16)]
            zw = zv[pl.ds(off, 16)]
            return (jnp.minimum(mn, yw), jnp.maximum(mx, yw),
                    sx + xw, sy + yw, sz + zw,
                    sxx + xw * xw, sxy + xw * yw, sxz + xw * zw,
                    syy + yw * yw, syz + yw * zw, szz + zw * zw)

        st = lax.fori_loop(0, nit, body1, init)
        for i in range(11):
            stv[i] = st[i]
        stv[11] = zero
        pltpu.sync_copy(stv, st_hbm.at[w])

        mn_s = jnp.min(st[0])
        mx_s = jnp.max(st[1])
        den_s = mx_s - mn_s + 1e-6
        mn_v = jnp.full((16,), mn_s, F32)
        den_v = jnp.full((16,), den_s, F32)

        for i in range(3):
            for rr in range(16):
                acc[i, rr] = zero
        lanes = lax.iota(jnp.int32, 16)
        idx0 = jnp.zeros((16,), jnp.int32)
        idx1 = idx0 + 1
        idx2 = idx0 + 2
        ones_f = jnp.ones((16,), F32)

        def body2(i, carry):
            off = i * 16
            y16 = yv[pl.ds(off, 16)]
            x16 = xv[pl.ds(off, 16)]
            z16 = zv[pl.ds(off, 16)]
            t = (y16 - mn_v) / den_v * 10.0
            bk = t.astype(jnp.int32)
            plsc.addupdate_scatter(acc, [idx0, bk, lanes], ones_f)
            plsc.addupdate_scatter(acc, [idx1, bk, lanes], x16)
            plsc.addupdate_scatter(acc, [idx2, bk, lanes], z16)
            return carry

        lax.fori_loop(0, nit, body2, 0)
        pltpu.sync_copy(acc, part_hbm.at[w])


_SC_MESH = dict(
    mesh=plsc.VectorSubcoreMesh(core_axis_name="c", subcore_axis_name="s"),
    compiler_params=pltpu.CompilerParams(needs_layout_passes=False),
)


def _sc_hist(xs_f, ys_f, zs_f):
    fn = functools.partial(
        pl.kernel,
        out_type=[jax.ShapeDtypeStruct((NBATCH, 3, 16, 16), F32),
                  jax.ShapeDtypeStruct((NBATCH, 12, 16), F32)],
        scratch_types=[
            pltpu.VMEM((N_PTS,), F32),
            pltpu.VMEM((N_PTS,), F32),
            pltpu.VMEM((N_PTS,), F32),
            pltpu.VMEM((3, 16, 16), F32),
            pltpu.VMEM((12, 16), F32),
        ],
        **_SC_MESH,
    )(_sc_body)
    return fn(xs_f, ys_f, zs_f)


def kernel(x, W1, b1, g1, be1, W2, b2, g2, be2, W3, b3, g3, be3,
           Wp, bp, ln_g, ln_b):
    B, N, _ = x.shape
    M = B * N
    minv = 1.0 / M
    grid = (B,)
    xspec = pl.BlockSpec((1, N, 3), lambda b_: (b_, 0, 0))

    def full(shp):
        return pl.BlockSpec(shp, lambda b_: (0,) * len(shp))

    def sds(shp):
        return jax.ShapeDtypeStruct(shp, F32)

    xs_f = x[:, :, 0].reshape(M)
    ys_f = x[:, :, 1].reshape(M)
    zs_f = x[:, :, 2].reshape(M)
    part, st = _sc_hist(xs_f, ys_f, zs_f)

    ymn_b = jnp.min(st[:, 0, :], axis=1)
    ymx_b = jnp.max(st[:, 1, :], axis=1)
    den_b = ymx_b - ymn_b + 1e-6
    mom = jnp.sum(st[:, 2:11, :], axis=(0, 2))
    mu3 = mom[0:3] * minv
    cov = (jnp.stack([
        jnp.stack([mom[3], mom[4], mom[5]]),
        jnp.stack([mom[4], mom[6], mom[7]]),
        jnp.stack([mom[5], mom[7], mom[8]]),
    ]) * minv - mu3[:, None] * mu3[None, :])
    m1 = mu3 @ W1.T + b1
    v1 = jnp.einsum("jc,cd,jd->j", W1, cov, W1)
    s1 = g1 / jnp.sqrt(v1 + 1e-5)
    a1 = W1.T * s1[None]
    c1 = ((b1 - m1) * s1 + be1)[None]

    agg = jnp.sum(part, axis=3)
    cnt = agg[:, 0]
    safe = jnp.maximum(cnt, 1.0)
    pos = cnt > 0.0
    cx = jnp.where(pos, agg[:, 1] / safe, 0.0)
    cz = jnp.where(pos, agg[:, 2] / safe, 0.0)
    ctr = jnp.concatenate([cx, cz], axis=1).reshape(B, 1, 32)
    prm = jnp.stack([ymn_b, den_b], axis=1).reshape(B, 1, 2)

    su2, sq2 = pl.pallas_call(
        _p2_body,
        grid=grid,
        in_specs=[xspec, full((3, 64)), full((1, 64)),
                  full((64, 128)), full((1, 128))],
        out_specs=[full((1, 128)), full((1, 128))],
        out_shape=[sds((1, 128)), sds((1, 128))],
    )(x, a1, c1, W2.T, b2[None])

    m2 = su2 * minv
    v2 = sq2 * minv - m2 * m2
    s2 = g2[None] / jnp.sqrt(v2 + 1e-5)
    a2 = W2.T * s2
    c2 = (b2[None] - m2) * s2 + be2[None]

    su3, sq3 = pl.pallas_call(
        _p3_body,
        grid=grid,
        in_specs=[xspec, full((3, 64)), full((1, 64)),
                  full((64, 128)), full((1, 128)),
                  full((128, 256)), full((1, 256))],
        out_specs=[full((1, 256)), full((1, 256))],
        out_shape=[sds((1, 256)), sds((1, 256))],
    )(x, a1, c1, a2, c2, W3.T, b3[None])

    m3 = su3 * minv
    v3 = sq3 * minv - m3 * m3
    s3 = g3[None] / jnp.sqrt(v3 + 1e-5)
    a3 = W3.T * s3
    c3 = (b3[None] - m3) * s3 + be3[None]

    xt = x.transpose(0, 2, 1)
    mxh, smh, mxs, sms = pl.pallas_call(
        _p4_body,
        grid=grid,
        in_specs=[xspec,
                  pl.BlockSpec((1, 3, N), lambda b_: (b_, 0, 0)),
                  full((3, 64)), full((1, 64)),
                  full((64, 128)), full((1, 128)),
                  full((128, 256)), full((1, 256)),
                  pl.BlockSpec((1, 1, 32), lambda b_: (b_, 0, 0),
                               memory_space=pltpu.SMEM),
                  pl.BlockSpec((1, 1, 2), lambda b_: (b_, 0, 0),
                               memory_space=pltpu.SMEM)],
        out_specs=[pl.BlockSpec((1, 1, 256), lambda b_: (b_, 0, 0)),
                   pl.BlockSpec((1, 1, 256), lambda b_: (b_, 0, 0)),
                   pl.BlockSpec((1, 1, 3), lambda b_: (b_, 0, 0)),
                   pl.BlockSpec((1, 1, 3), lambda b_: (b_, 0, 0))],
        out_shape=[sds((B, 1, 256)), sds((B, 1, 256)),
                   sds((B, 1, 3)), sds((B, 1, 3))],
    )(x, xt, a1, c1, a2, c2, a3, c3, ctr, prm)

    wpt = Wp.T
    out = pl.pallas_call(
        _p5_body,
        out_shape=sds((B, 512)),
    )(mxh.reshape(B, 256), mxs.reshape(B, 3),
      smh.reshape(B, 256), sms.reshape(B, 3),
      wpt[0:256], wpt[256:259], wpt[259:515], wpt[515:518],
      bp[None], ln_g[None], ln_b[None])
    return out


# speedup vs baseline: 1.0759x; 1.0665x over previous
import functools

import jax
import jax.numpy as jnp
from jax import lax
from jax.experimental import pallas as pl
from jax.experimental.pallas import tpu as pltpu
from jax.experimental.pallas import tpu_sc as plsc

F32 = jnp.float32
N_PTS = 16384
NBATCH = 16


def _p2_body(x_ref, a1_ref, c1_ref, w_ref, b_ref, su_ref, sq_ref):
    bi = pl.program_id(0)
    xb = x_ref[0]
    h1 = jnp.maximum(
        jnp.dot(xb, a1_ref[...], preferred_element_type=F32) + c1_ref[...], 0.0)
    u = jnp.dot(h1.astype(jnp.bfloat16), w_ref[...],
                preferred_element_type=F32) + b_ref[...]
    su = jnp.sum(u, axis=0, keepdims=True)
    sq = jnp.sum(u * u, axis=0, keepdims=True)

    @pl.when(bi == 0)
    def _():
        su_ref[...] = su
        sq_ref[...] = sq

    @pl.when(bi != 0)
    def _():
        su_ref[...] += su
        sq_ref[...] += sq


def _p3_body(x_ref, a1_ref, c1_ref, a2_ref, c2_ref, w_ref, b_ref,
             su_ref, sq_ref):
    bi = pl.program_id(0)
    xb = x_ref[0]
    h1 = jnp.maximum(
        jnp.dot(xb, a1_ref[...], preferred_element_type=F32) + c1_ref[...], 0.0)
    h2 = jnp.maximum(
        jnp.dot(h1.astype(jnp.bfloat16), a2_ref[...],
                preferred_element_type=F32) + c2_ref[...], 0.0)
    u = jnp.dot(h2.astype(jnp.bfloat16), w_ref[...],
                preferred_element_type=F32) + b_ref[...]
    su = jnp.sum(u, axis=0, keepdims=True)
    sq = jnp.sum(u * u, axis=0, keepdims=True)

    @pl.when(bi == 0)
    def _():
        su_ref[...] = su
        sq_ref[...] = sq

    @pl.when(bi != 0)
    def _():
        su_ref[...] += su
        sq_ref[...] += sq


def _p4_body(x_ref, xt_ref, a1_ref, c1_ref, a2_ref, c2_ref, a3_ref, c3_ref,
             ctr_ref, prm_ref,
             mxh_ref, smh_ref, mxs_ref, sms_ref):
    xb = x_ref[0]
    h1 = jnp.maximum(
        jnp.dot(xb, a1_ref[...], preferred_element_type=F32) + c1_ref[...], 0.0)
    h2 = jnp.maximum(
        jnp.dot(h1, a2_ref[...], preferred_element_type=F32) + c2_ref[...], 0.0)
    h3 = jnp.maximum(
        jnp.dot(h2, a3_ref[...], preferred_element_type=F32) + c3_ref[...], 0.0)
    mxh = jnp.max(h3, axis=0, keepdims=True)
    smh = jnp.sum(h3, axis=0, keepdims=True)

    xtb = xt_ref[0]
    xs = xtb[0:1]
    yv = xtb[1:2]
    zs = xtb[2:3]
    t = (yv - prm_ref[0, 0, 0]) / prm_ref[0, 0, 1] * 10.0
    bk = t.astype(jnp.int32)
    cx = jnp.zeros((1, N_PTS), F32)
    cz = jnp.zeros((1, N_PTS), F32)
    for k in range(11):
        mk = bk == k
        cx = jnp.where(mk, ctr_ref[0, 0, k], cx)
        cz = jnp.where(mk, ctr_ref[0, 0, 16 + k], cz)
    dx = xs - cx
    dz = zs - cz
    r = jnp.sqrt(dx * dx + dz * dz)
    inv = jnp.where(r > 0.0, 1.0 / r, 0.0)
    sn = dz * inv
    cs = jnp.where(r > 0.0, dx * inv, 1.0)

    def c3_(a, b, c):
        return jnp.concatenate(
            [jnp.reshape(a, (1, 1)), jnp.reshape(b, (1, 1)),
             jnp.reshape(c, (1, 1))], axis=1)

    mxs = c3_(jnp.max(sn), jnp.max(cs), jnp.max(r))
    sms = c3_(jnp.sum(sn), jnp.sum(cs), jnp.sum(r))

    mxh_ref[...] = mxh[None]
    smh_ref[...] = smh[None]
    mxs_ref[...] = mxs[None]
    sms_ref[...] = sms[None]


def _p5_body(mxh_ref, mxs_ref, smh_ref, sms_ref,
             w1_ref, w2_ref, w3_ref, w4_ref,
             bp_ref, g_ref, b_ref, out_ref):
    invn = 1.0 / N_PTS
    o = (jnp.dot(mxh_ref[...], w1_ref[...], preferred_element_type=F32)
         + jnp.dot(mxs_ref[...], w2_ref[...], preferred_element_type=F32)
         + jnp.dot(smh_ref[...] * invn, w3_ref[...], preferred_element_type=F32)
         + jnp.dot(sms_ref[...] * invn, w4_ref[...], preferred_element_type=F32)
         + bp_ref[...])
    mu = jnp.mean(o, axis=1, keepdims=True)
    var = jnp.mean((o - mu) ** 2, axis=1, keepdims=True)
    out_ref[...] = (o - mu) / jnp.sqrt(var + 1e-5) * g_ref[...] + b_ref[...]


def _sc_body(xs_hbm, ys_hbm, zs_hbm, part_hbm, st_hbm, xv, yv, zv, acc, stv):
    c = lax.axis_index("c")
    s = lax.axis_index("s")
    w = s * 2 + c

    @pl.when(w < NBATCH)
    def _():
        base = w * N_PTS
        pltpu.sync_copy(xs_hbm.at[pl.ds(base, N_PTS)], xv)
        pltpu.sync_copy(ys_hbm.at[pl.ds(base, N_PTS)], yv)
        pltpu.sync_copy(zs_hbm.at[pl.ds(base, N_PTS)], zv)
        nit = N_PTS // 16
        big = jnp.float32(3.4e38)
        zero = jnp.zeros((16,), F32)
        init = (jnp.full((16,), big, F32), jnp.full((16,), -big, F32),
                zero, zero, zero, zero, zero, zero, zero, zero, zero)

        def body1(i, cr):
            mn, mx, sx, sy, sz, sxx, sxy, sxz, syy, syz, szz = cr
            off = i * 16
            xw = xv[pl.ds(off, 16)]
            yw = yv[pl.ds(off, 16)]
            zw = zv[pl.ds(off, 16)]
            return (jnp.minimum(mn, yw), jnp.maximum(mx, yw),
                    sx + xw, sy + yw, sz + zw,
                    sxx + xw * xw, sxy + xw * yw, sxz + xw * zw,
                    syy + yw * yw, syz + yw * zw, szz + zw * zw)

        st = lax.fori_loop(0, nit, body1, init)
        for i in range(11):
            stv[i] = st[i]
        stv[11] = zero
        pltpu.sync_copy(stv, st_hbm.at[w])

        mn_s = jnp.min(st[0])
        mx_s = jnp.max(st[1])
        den_s = mx_s - mn_s + 1e-6
        mn_v = jnp.full((16,), mn_s, F32)
        den_v = jnp.full((16,), den_s, F32)

        for i in range(3):
            for rr in range(16):
                acc[i, rr] = zero
        lanes = lax.iota(jnp.int32, 16)
        idx0 = jnp.zeros((16,), jnp.int32)
        idx1 = idx0 + 1
        idx2 = idx0 + 2
        ones_f = jnp.ones((16,), F32)

        def body2(i, carry):
            off = i * 16
            y16 = yv[pl.ds(off, 16)]
            x16 = xv[pl.ds(off, 16)]
            z16 = zv[pl.ds(off, 16)]
            t = (y16 - mn_v) / den_v * 10.0
            bk = t.astype(jnp.int32)
            plsc.addupdate_scatter(acc, [idx0, bk, lanes], ones_f)
            plsc.addupdate_scatter(acc, [idx1, bk, lanes], x16)
            plsc.addupdate_scatter(acc, [idx2, bk, lanes], z16)
            return carry

        lax.fori_loop(0, nit, body2, 0)
        pltpu.sync_copy(acc, part_hbm.at[w])


_SC_MESH = dict(
    mesh=plsc.VectorSubcoreMesh(core_axis_name="c", subcore_axis_name="s"),
    compiler_params=pltpu.CompilerParams(needs_layout_passes=False),
)


def _sc_hist(xs_f, ys_f, zs_f):
    fn = functools.partial(
        pl.kernel,
        out_type=[jax.ShapeDtypeStruct((NBATCH, 3, 16, 16), F32),
                  jax.ShapeDtypeStruct((NBATCH, 12, 16), F32)],
        scratch_types=[
            pltpu.VMEM((N_PTS,), F32),
            pltpu.VMEM((N_PTS,), F32),
            pltpu.VMEM((N_PTS,), F32),
            pltpu.VMEM((3, 16, 16), F32),
            pltpu.VMEM((12, 16), F32),
        ],
        **_SC_MESH,
    )(_sc_body)
    return fn(xs_f, ys_f, zs_f)


def kernel(x, W1, b1, g1, be1, W2, b2, g2, be2, W3, b3, g3, be3,
           Wp, bp, ln_g, ln_b):
    B, N, _ = x.shape
    M = B * N
    minv = 1.0 / M
    grid = (B,)
    xspec = pl.BlockSpec((1, N, 3), lambda b_: (b_, 0, 0))

    def full(shp):
        return pl.BlockSpec(shp, lambda b_: (0,) * len(shp))

    def sds(shp):
        return jax.ShapeDtypeStruct(shp, F32)

    xs_f = x[:, :, 0].reshape(M)
    ys_f = x[:, :, 1].reshape(M)
    zs_f = x[:, :, 2].reshape(M)
    part, st = _sc_hist(xs_f, ys_f, zs_f)

    ymn_b = jnp.min(st[:, 0, :], axis=1)
    ymx_b = jnp.max(st[:, 1, :], axis=1)
    den_b = ymx_b - ymn_b + 1e-6
    mom = jnp.sum(st[:, 2:11, :], axis=(0, 2))
    mu3 = mom[0:3] * minv
    cov = (jnp.stack([
        jnp.stack([mom[3], mom[4], mom[5]]),
        jnp.stack([mom[4], mom[6], mom[7]]),
        jnp.stack([mom[5], mom[7], mom[8]]),
    ]) * minv - mu3[:, None] * mu3[None, :])
    m1 = mu3 @ W1.T + b1
    v1 = jnp.einsum("jc,cd,jd->j", W1, cov, W1)
    s1 = g1 / jnp.sqrt(v1 + 1e-5)
    a1 = W1.T * s1[None]
    c1 = ((b1 - m1) * s1 + be1)[None]

    agg = jnp.sum(part, axis=3)
    cnt = agg[:, 0]
    safe = jnp.maximum(cnt, 1.0)
    pos = cnt > 0.0
    cx = jnp.where(pos, agg[:, 1] / safe, 0.0)
    cz = jnp.where(pos, agg[:, 2] / safe, 0.0)
    ctr = jnp.concatenate([cx, cz], axis=1).reshape(B, 1, 32)
    prm = jnp.stack([ymn_b, den_b], axis=1).reshape(B, 1, 2)

    su2, sq2 = pl.pallas_call(
        _p2_body,
        grid=grid,
        in_specs=[xspec, full((3, 64)), full((1, 64)),
                  full((64, 128)), full((1, 128))],
        out_specs=[full((1, 128)), full((1, 128))],
        out_shape=[sds((1, 128)), sds((1, 128))],
    )(x, a1, c1, W2.T.astype(jnp.bfloat16), b2[None])

    m2 = su2 * minv
    v2 = sq2 * minv - m2 * m2
    s2 = g2[None] / jnp.sqrt(v2 + 1e-5)
    a2 = W2.T * s2
    c2 = (b2[None] - m2) * s2 + be2[None]

    su3, sq3 = pl.pallas_call(
        _p3_body,
        grid=grid,
        in_specs=[xspec, full((3, 64)), full((1, 64)),
                  full((64, 128)), full((1, 128)),
                  full((128, 256)), full((1, 256))],
        out_specs=[full((1, 256)), full((1, 256))],
        out_shape=[sds((1, 256)), sds((1, 256))],
    )(x, a1, c1, a2.astype(jnp.bfloat16), c2,
      W3.T.astype(jnp.bfloat16), b3[None])

    m3 = su3 * minv
    v3 = sq3 * minv - m3 * m3
    s3 = g3[None] / jnp.sqrt(v3 + 1e-5)
    a3 = W3.T * s3
    c3 = (b3[None] - m3) * s3 + be3[None]

    xt = x.transpose(0, 2, 1)
    mxh, smh, mxs, sms = pl.pallas_call(
        _p4_body,
        grid=grid,
        in_specs=[xspec,
                  pl.BlockSpec((1, 3, N), lambda b_: (b_, 0, 0)),
                  full((3, 64)), full((1, 64)),
                  full((64, 128)), full((1, 128)),
                  full((128, 256)), full((1, 256)),
                  pl.BlockSpec((1, 1, 32), lambda b_: (b_, 0, 0),
                               memory_space=pltpu.SMEM),
                  pl.BlockSpec((1, 1, 2), lambda b_: (b_, 0, 0),
                               memory_space=pltpu.SMEM)],
        out_specs=[pl.BlockSpec((1, 1, 256), lambda b_: (b_, 0, 0)),
                   pl.BlockSpec((1, 1, 256), lambda b_: (b_, 0, 0)),
                   pl.BlockSpec((1, 1, 3), lambda b_: (b_, 0, 0)),
                   pl.BlockSpec((1, 1, 3), lambda b_: (b_, 0, 0))],
        out_shape=[sds((B, 1, 256)), sds((B, 1, 256)),
                   sds((B, 1, 3)), sds((B, 1, 3))],
    )(x, xt, a1, c1, a2, c2, a3, c3, ctr, prm)

    wpt = Wp.T
    out = pl.pallas_call(
        _p5_body,
        out_shape=sds((B, 512)),
    )(mxh.reshape(B, 256), mxs.reshape(B, 3),
      smh.reshape(B, 256), sms.reshape(B, 3),
      wpt[0:256], wpt[256:259], wpt[259:515], wpt[515:518],
      bp[None], ln_g[None], ln_b[None])
    return out


# speedup vs baseline: 1.1613x; 1.0794x over previous
import functools

import jax
import jax.numpy as jnp
from jax import lax
from jax.experimental import pallas as pl
from jax.experimental.pallas import tpu as pltpu
from jax.experimental.pallas import tpu_sc as plsc

F32 = jnp.float32
N_PTS = 16384
NBATCH = 16


def _p2_body(x_ref, a1_ref, c1_ref, su_ref, sq_ref):
    bi = pl.program_id(0)
    xb = x_ref[0]
    h1 = jnp.maximum(
        jnp.dot(xb, a1_ref[...], preferred_element_type=F32) + c1_ref[...], 0.0)
    su = jnp.sum(h1, axis=0, keepdims=True)
    sq = lax.dot_general(h1, h1, (((0,), (0,)), ((), ())),
                         preferred_element_type=F32)

    @pl.when(bi == 0)
    def _():
        su_ref[...] = su
        sq_ref[...] = sq

    @pl.when(bi != 0)
    def _():
        su_ref[...] += su
        sq_ref[...] += sq


def _p3_body(x_ref, a1_ref, c1_ref, a2_ref, c2_ref, su_ref, sq_ref):
    bi = pl.program_id(0)
    xb = x_ref[0]
    h1 = jnp.maximum(
        jnp.dot(xb, a1_ref[...], preferred_element_type=F32) + c1_ref[...], 0.0)
    h2 = jnp.maximum(
        jnp.dot(h1.astype(jnp.bfloat16), a2_ref[...],
                preferred_element_type=F32) + c2_ref[...], 0.0)
    su = jnp.sum(h2, axis=0, keepdims=True)
    sq = lax.dot_general(h2, h2, (((0,), (0,)), ((), ())),
                         preferred_element_type=F32)

    @pl.when(bi == 0)
    def _():
        su_ref[...] = su
        sq_ref[...] = sq

    @pl.when(bi != 0)
    def _():
        su_ref[...] += su
        sq_ref[...] += sq


def _p4_body(x_ref, xt_ref, a1_ref, c1_ref, a2_ref, c2_ref, a3_ref, c3_ref,
             ctr_ref, prm_ref,
             mxh_ref, smh_ref, mxs_ref, sms_ref):
    xb = x_ref[0]
    h1 = jnp.maximum(
        jnp.dot(xb, a1_ref[...], preferred_element_type=F32) + c1_ref[...], 0.0)
    h2 = jnp.maximum(
        jnp.dot(h1, a2_ref[...], preferred_element_type=F32) + c2_ref[...], 0.0)
    h3 = jnp.maximum(
        jnp.dot(h2, a3_ref[...], preferred_element_type=F32) + c3_ref[...], 0.0)
    mxh = jnp.max(h3, axis=0, keepdims=True)
    smh = jnp.sum(h3, axis=0, keepdims=True)

    xtb = xt_ref[0]
    xs = xtb[0:1]
    yv = xtb[1:2]
    zs = xtb[2:3]
    t = (yv - prm_ref[0, 0, 0]) / prm_ref[0, 0, 1] * 10.0
    bk = t.astype(jnp.int32)
    cx = jnp.zeros((1, N_PTS), F32)
    cz = jnp.zeros((1, N_PTS), F32)
    for k in range(11):
        mk = bk == k
        cx = jnp.where(mk, ctr_ref[0, 0, k], cx)
        cz = jnp.where(mk, ctr_ref[0, 0, 16 + k], cz)
    dx = xs - cx
    dz = zs - cz
    r = jnp.sqrt(dx * dx + dz * dz)
    inv = jnp.where(r > 0.0, 1.0 / r, 0.0)
    sn = dz * inv
    cs = jnp.where(r > 0.0, dx * inv, 1.0)

    def c3_(a, b, c):
        return jnp.concatenate(
            [jnp.reshape(a, (1, 1)), jnp.reshape(b, (1, 1)),
             jnp.reshape(c, (1, 1))], axis=1)

    mxs = c3_(jnp.max(sn), jnp.max(cs), jnp.max(r))
    sms = c3_(jnp.sum(sn), jnp.sum(cs), jnp.sum(r))

    mxh_ref[...] = mxh[None]
    smh_ref[...] = smh[None]
    mxs_ref[...] = mxs[None]
    sms_ref[...] = sms[None]


def _p5_body(mxh_ref, mxs_ref, smh_ref, sms_ref,
             w1_ref, w2_ref, w3_ref, w4_ref,
             bp_ref, g_ref, b_ref, out_ref):
    invn = 1.0 / N_PTS
    o = (jnp.dot(mxh_ref[...], w1_ref[...], preferred_element_type=F32)
         + jnp.dot(mxs_ref[...], w2_ref[...], preferred_element_type=F32)
         + jnp.dot(smh_ref[...] * invn, w3_ref[...], preferred_element_type=F32)
         + jnp.dot(sms_ref[...] * invn, w4_ref[...], preferred_element_type=F32)
         + bp_ref[...])
    mu = jnp.mean(o, axis=1, keepdims=True)
    var = jnp.mean((o - mu) ** 2, axis=1, keepdims=True)
    out_ref[...] = (o - mu) / jnp.sqrt(var + 1e-5) * g_ref[...] + b_ref[...]


def _sc_body(xs_hbm, ys_hbm, zs_hbm, part_hbm, st_hbm, xv, yv, zv, acc, stv):
    c = lax.axis_index("c")
    s = lax.axis_index("s")
    w = s * 2 + c

    @pl.when(w < NBATCH)
    def _():
        base = w * N_PTS
        pltpu.sync_copy(xs_hbm.at[pl.ds(base, N_PTS)], xv)
        pltpu.sync_copy(ys_hbm.at[pl.ds(base, N_PTS)], yv)
        pltpu.sync_copy(zs_hbm.at[pl.ds(base, N_PTS)], zv)
        nit = N_PTS // 16
        big = jnp.float32(3.4e38)
        zero = jnp.zeros((16,), F32)
        init = (jnp.full((16,), big, F32), jnp.full((16,), -big, F32),
                zero, zero, zero, zero, zero, zero, zero, zero, zero)

        def body1(i, cr):
            mn, mx, sx, sy, sz, sxx, sxy, sxz, syy, syz, szz = cr
            off = i * 16
            xw = xv[pl.ds(off, 16)]
            yw = yv[pl.ds(off, 16)]
            zw = zv[pl.ds(off, 16)]
            return (jnp.minimum(mn, yw), jnp.maximum(mx, yw),
                    sx + xw, sy + yw, sz + zw,
                    sxx + xw * xw, sxy + xw * yw, sxz + xw * zw,
                    syy + yw * yw, syz + yw * zw, szz + zw * zw)

        st = lax.fori_loop(0, nit, body1, init)
        for i in range(11):
            stv[i] = st[i]
        stv[11] = zero
        pltpu.sync_copy(stv, st_hbm.at[w])

        mn_s = jnp.min(st[0])
        mx_s = jnp.max(st[1])
        den_s = mx_s - mn_s + 1e-6
        mn_v = jnp.full((16,), mn_s, F32)
        den_v = jnp.full((16,), den_s, F32)

        for i in range(3):
            for rr in range(16):
                acc[i, rr] = zero
        lanes = lax.iota(jnp.int32, 16)
        idx0 = jnp.zeros((16,), jnp.int32)
        idx1 = idx0 + 1
        idx2 = idx0 + 2
        ones_f = jnp.ones((16,), F32)

        def body2(i, carry):
            off = i * 16
            y16 = yv[pl.ds(off, 16)]
            x16 = xv[pl.ds(off, 16)]
            z16 = zv[pl.ds(off, 16)]
            t = (y16 - mn_v) / den_v * 10.0
            bk = t.astype(jnp.int32)
            plsc.addupdate_scatter(acc, [idx0, bk, lanes], ones_f)
            plsc.addupdate_scatter(acc, [idx1, bk, lanes], x16)
            plsc.addupdate_scatter(acc, [idx2, bk, lanes], z16)
            return carry

        lax.fori_loop(0, nit, body2, 0)
        pltpu.sync_copy(acc, part_hbm.at[w])


_SC_MESH = dict(
    mesh=plsc.VectorSubcoreMesh(core_axis_name="c", subcore_axis_name="s"),
    compiler_params=pltpu.CompilerParams(needs_layout_passes=False),
)


def _sc_hist(xs_f, ys_f, zs_f):
    fn = functools.partial(
        pl.kernel,
        out_type=[jax.ShapeDtypeStruct((NBATCH, 3, 16, 16), F32),
                  jax.ShapeDtypeStruct((NBATCH, 12, 16), F32)],
        scratch_types=[
            pltpu.VMEM((N_PTS,), F32),
            pltpu.VMEM((N_PTS,), F32),
            pltpu.VMEM((N_PTS,), F32),
            pltpu.VMEM((3, 16, 16), F32),
            pltpu.VMEM((12, 16), F32),
        ],
        **_SC_MESH,
    )(_sc_body)
    return fn(xs_f, ys_f, zs_f)


def kernel(x, W1, b1, g1, be1, W2, b2, g2, be2, W3, b3, g3, be3,
           Wp, bp, ln_g, ln_b):
    B, N, _ = x.shape
    M = B * N
    minv = 1.0 / M
    grid = (B,)
    xspec = pl.BlockSpec((1, N, 3), lambda b_: (b_, 0, 0))

    def full(shp):
        return pl.BlockSpec(shp, lambda b_: (0,) * len(shp))

    def sds(shp):
        return jax.ShapeDtypeStruct(shp, F32)

    xs_f = x[:, :, 0].reshape(M)
    ys_f = x[:, :, 1].reshape(M)
    zs_f = x[:, :, 2].reshape(M)
    part, st = _sc_hist(xs_f, ys_f, zs_f)

    ymn_b = jnp.min(st[:, 0, :], axis=1)
    ymx_b = jnp.max(st[:, 1, :], axis=1)
    den_b = ymx_b - ymn_b + 1e-6
    mom = jnp.sum(st[:, 2:11, :], axis=(0, 2))
    mu3 = mom[0:3] * minv
    cov = (jnp.stack([
        jnp.stack([mom[3], mom[4], mom[5]]),
        jnp.stack([mom[4], mom[6], mom[7]]),
        jnp.stack([mom[5], mom[7], mom[8]]),
    ]) * minv - mu3[:, None] * mu3[None, :])
    m1 = mu3 @ W1.T + b1
    v1 = jnp.einsum("jc,cd,jd->j", W1, cov, W1)
    s1 = g1 / jnp.sqrt(v1 + 1e-5)
    a1 = W1.T * s1[None]
    c1 = ((b1 - m1) * s1 + be1)[None]

    agg = jnp.sum(part, axis=3)
    cnt = agg[:, 0]
    safe = jnp.maximum(cnt, 1.0)
    pos = cnt > 0.0
    cx = jnp.where(pos, agg[:, 1] / safe, 0.0)
    cz = jnp.where(pos, agg[:, 2] / safe, 0.0)
    ctr = jnp.concatenate([cx, cz], axis=1).reshape(B, 1, 32)
    prm = jnp.stack([ymn_b, den_b], axis=1).reshape(B, 1, 2)

    sh1, gh1 = pl.pallas_call(
        _p2_body,
        grid=grid,
        in_specs=[xspec, full((3, 64)), full((1, 64))],
        out_specs=[full((1, 64)), full((64, 64))],
        out_shape=[sds((1, 64)), sds((64, 64))],
    )(x, a1, c1)

    mh1 = sh1 * minv
    ch1 = gh1 * minv - mh1.T @ mh1
    m2 = mh1 @ W2.T + b2[None]
    v2 = jnp.einsum("jc,cd,jd->j", W2, ch1, W2)[None]
    s2 = g2[None] / jnp.sqrt(v2 + 1e-5)
    a2 = W2.T * s2
    c2 = (b2[None] - m2) * s2 + be2[None]

    sh2, gh2 = pl.pallas_call(
        _p3_body,
        grid=grid,
        in_specs=[xspec, full((3, 64)), full((1, 64)),
                  full((64, 128)), full((1, 128))],
        out_specs=[full((1, 128)), full((128, 128))],
        out_shape=[sds((1, 128)), sds((128, 128))],
    )(x, a1, c1, a2.astype(jnp.bfloat16), c2)

    mh2 = sh2 * minv
    ch2 = gh2 * minv - mh2.T @ mh2
    m3 = mh2 @ W3.T + b3[None]
    v3 = jnp.einsum("jc,cd,jd->j", W3, ch2, W3)[None]
    s3 = g3[None] / jnp.sqrt(v3 + 1e-5)
    a3 = W3.T * s3
    c3 = (b3[None] - m3) * s3 + be3[None]

    xt = x.transpose(0, 2, 1)
    mxh, smh, mxs, sms = pl.pallas_call(
        _p4_body,
        grid=grid,
        in_specs=[xspec,
                  pl.BlockSpec((1, 3, N), lambda b_: (b_, 0, 0)),
                  full((3, 64)), full((1, 64)),
                  full((64, 128)), full((1, 128)),
                  full((128, 256)), full((1, 256)),
                  pl.BlockSpec((1, 1, 32), lambda b_: (b_, 0, 0),
                               memory_space=pltpu.SMEM),
                  pl.BlockSpec((1, 1, 2), lambda b_: (b_, 0, 0),
                               memory_space=pltpu.SMEM)],
        out_specs=[pl.BlockSpec((1, 1, 256), lambda b_: (b_, 0, 0)),
                   pl.BlockSpec((1, 1, 256), lambda b_: (b_, 0, 0)),
                   pl.BlockSpec((1, 1, 3), lambda b_: (b_, 0, 0)),
                   pl.BlockSpec((1, 1, 3), lambda b_: (b_, 0, 0))],
        out_shape=[sds((B, 1, 256)), sds((B, 1, 256)),
                   sds((B, 1, 3)), sds((B, 1, 3))],
    )(x, xt, a1, c1, a2, c2, a3, c3, ctr, prm)

    wpt = Wp.T
    out = pl.pallas_call(
        _p5_body,
        out_shape=sds((B, 512)),
    )(mxh.reshape(B, 256), mxs.reshape(B, 3),
      smh.reshape(B, 256), sms.reshape(B, 3),
      wpt[0:256], wpt[256:259], wpt[259:515], wpt[515:518],
      bp[None], ln_g[None], ln_b[None])
    return out
